# Initial kernel scaffold; baseline (speedup 1.0000x reference)
import functools

import jax
import jax.numpy as jnp
from jax import lax
from jax.experimental import pallas as pl
from jax.experimental.pallas import tpu as pltpu
from jax.experimental.pallas import tpu_sc as plsc

N = 10000
E = 320000
D = 128

NC = 2
NS = 16
NW = NC * NS
CHUNK = 128
NCHUNKS = E // CHUNK
ROWS_PT = N // NS
LANES = 16


_ROWS_BLK = 2000


def _pre_body(x_ref, wn_ref, ws_ref, b_ref, y_ref, h_ref):
    x = x_ref[...]
    y_ref[...] = lax.dot_general(
        x, wn_ref[...], (((1,), (1,)), ((), ())),
        preferred_element_type=jnp.float32)
    h_ref[...] = lax.dot_general(
        x, ws_ref[...], (((1,), (1,)), ((), ())),
        preferred_element_type=jnp.float32) + b_ref[...]


@jax.jit
def _pre(x, W_neigh, W_self, b2d):
    return pl.pallas_call(
        _pre_body,
        grid=(N // _ROWS_BLK,),
        in_specs=[
            pl.BlockSpec((_ROWS_BLK, D), lambda i: (i, 0)),
            pl.BlockSpec((D, D), lambda i: (0, 0)),
            pl.BlockSpec((D, D), lambda i: (0, 0)),
            pl.BlockSpec((1, D), lambda i: (0, 0)),
        ],
        out_specs=[
            pl.BlockSpec((_ROWS_BLK, D), lambda i: (i, 0)),
            pl.BlockSpec((_ROWS_BLK, D), lambda i: (i, 0)),
        ],
        out_shape=[
            jax.ShapeDtypeStruct((N, D), jnp.float32),
            jax.ShapeDtypeStruct((N, D), jnp.float32),
        ],
    )(x, W_neigh, W_self, b2d)


_sc_mesh = plsc.VectorSubcoreMesh(core_axis_name="c", subcore_axis_name="s")


@functools.partial(
    pl.kernel,
    out_type=(
        jax.ShapeDtypeStruct((NC, N, D), jnp.float32),
        jax.ShapeDtypeStruct((NC, N), jnp.float32),
    ),
    mesh=_sc_mesh,
    scratch_types=[
        pltpu.VMEM((CHUNK,), jnp.int32),
        pltpu.VMEM((CHUNK,), jnp.int32),
        pltpu.VMEM((CHUNK, D), jnp.float32),
        pltpu.VMEM((CHUNK,), jnp.float32),
        pltpu.VMEM_SHARED((N, D), jnp.float32),
        pltpu.VMEM_SHARED((N,), jnp.float32),
        pltpu.SemaphoreType.DMA,
    ],
)
def _sc_seg(y_hbm, src_hbm, dst_hbm, z2_hbm, z1_hbm, seg_out, deg_out,
            src_v, dst_v, rows_v, ones_v, acc_sh, deg_sh, sem):
    c = lax.axis_index("c")
    s = lax.axis_index("s")
    w = s * NC + c

    pltpu.sync_copy(z2_hbm.at[pl.ds(s * ROWS_PT, ROWS_PT)],
                    acc_sh.at[pl.ds(s * ROWS_PT, ROWS_PT)])

    @pl.when(s == 0)
    def _():
        pltpu.sync_copy(z1_hbm, deg_sh)

    for i in range(CHUNK // LANES):
        ones_v[pl.ds(i * LANES, LANES)] = jnp.ones((LANES,), jnp.float32)

    plsc.subcore_barrier()

    n_full = NCHUNKS // NW
    n_j = n_full + jnp.where(w < NCHUNKS - n_full * NW, 1, 0)

    def body(j, carry):
        base = (j * NW + w) * CHUNK
        pltpu.sync_copy(src_hbm.at[pl.ds(base, CHUNK)], src_v)
        pltpu.sync_copy(dst_hbm.at[pl.ds(base, CHUNK)], dst_v)
        pltpu.async_copy(y_hbm.at[src_v], rows_v, sem).wait()
        pltpu.sync_copy(rows_v, acc_sh.at[dst_v], add=True)
        pltpu.sync_copy(ones_v, deg_sh.at[dst_v], add=True)
        return carry

    lax.fori_loop(0, n_j, body, 0)

    plsc.subcore_barrier()

    pltpu.sync_copy(acc_sh.at[pl.ds(s * ROWS_PT, ROWS_PT)],
                    seg_out.at[c, pl.ds(s * ROWS_PT, ROWS_PT)])

    @pl.when(s == 0)
    def _():
        pltpu.sync_copy(deg_sh, deg_out.at[c])


def _post_body(h_ref, seg_ref, deg_ref, o_ref):
    ssum = seg_ref[0] + seg_ref[1]
    dsum = deg_ref[0] + deg_ref[1]
    o_ref[...] = h_ref[...] + ssum / jnp.maximum(dsum, 1.0)


@jax.jit
def _post(h_self, seg, deg3):
    return pl.pallas_call(
        _post_body,
        grid=(N // _ROWS_BLK,),
        in_specs=[
            pl.BlockSpec((_ROWS_BLK, D), lambda i: (i, 0)),
            pl.BlockSpec((NC, _ROWS_BLK, D), lambda i: (0, i, 0)),
            pl.BlockSpec((NC, _ROWS_BLK, 1), lambda i: (0, i, 0)),
        ],
        out_specs=pl.BlockSpec((_ROWS_BLK, D), lambda i: (i, 0)),
        out_shape=jax.ShapeDtypeStruct((N, D), jnp.float32),
    )(h_self, seg, deg3)


def kernel(x, edge_index, W_neigh, W_self, b_self):
    src = edge_index[0]
    dst = edge_index[1]
    y, h_self = _pre(x, W_neigh, W_self, b_self.reshape(1, D))
    z2 = jnp.zeros((N, D), jnp.float32)
    z1 = jnp.zeros((N,), jnp.float32)
    seg, deg = _sc_seg(y, src, dst, z2, z1)
    return _post(h_self, seg, deg.reshape(NC, N, 1))


# speedup vs baseline: 7.2627x; 7.2627x over previous
import functools

import jax
import jax.numpy as jnp
from jax import lax
from jax.experimental import pallas as pl
from jax.experimental.pallas import tpu as pltpu
from jax.experimental.pallas import tpu_sc as plsc

N = 10000
E = 320000
D = 128

NC = 2
NS = 16
NW = NC * NS
CHUNK = 128
NCHUNKS = E // CHUNK
ROWS_PT = 624
ROWS_TAIL = N - NS * ROWS_PT
LANES = 16


_ROWS_BLK = 2000


def _pre_body(x_ref, wn_ref, ws_ref, b_ref, y_ref, h_ref):
    x = x_ref[...]
    y_ref[...] = lax.dot_general(
        x, wn_ref[...], (((1,), (1,)), ((), ())),
        preferred_element_type=jnp.float32)
    h_ref[...] = lax.dot_general(
        x, ws_ref[...], (((1,), (1,)), ((), ())),
        preferred_element_type=jnp.float32) + b_ref[...]


@jax.jit
def _pre(x, W_neigh, W_self, b2d):
    return pl.pallas_call(
        _pre_body,
        grid=(N // _ROWS_BLK,),
        in_specs=[
            pl.BlockSpec((_ROWS_BLK, D), lambda i: (i, 0)),
            pl.BlockSpec((D, D), lambda i: (0, 0)),
            pl.BlockSpec((D, D), lambda i: (0, 0)),
            pl.BlockSpec((1, D), lambda i: (0, 0)),
        ],
        out_specs=[
            pl.BlockSpec((_ROWS_BLK, D), lambda i: (i, 0)),
            pl.BlockSpec((_ROWS_BLK, D), lambda i: (i, 0)),
        ],
        out_shape=[
            jax.ShapeDtypeStruct((N, D), jnp.float32),
            jax.ShapeDtypeStruct((N, D), jnp.float32),
        ],
    )(x, W_neigh, W_self, b2d)


_sc_mesh = plsc.VectorSubcoreMesh(core_axis_name="c", subcore_axis_name="s")


@functools.partial(
    pl.kernel,
    out_type=(
        jax.ShapeDtypeStruct((NC, N, D), jnp.float32),
        jax.ShapeDtypeStruct((NC, N), jnp.float32),
    ),
    mesh=_sc_mesh,
    scratch_types=[
        pltpu.VMEM((CHUNK,), jnp.int32),
        pltpu.VMEM((CHUNK,), jnp.int32),
        pltpu.VMEM((CHUNK, D), jnp.float32),
        pltpu.VMEM((CHUNK,), jnp.float32),
        pltpu.VMEM_SHARED((N, D), jnp.float32),
        pltpu.VMEM_SHARED((N,), jnp.float32),
        pltpu.SemaphoreType.DMA,
    ],
)
def _sc_seg(y_hbm, src_hbm, dst_hbm, z2_hbm, z1_hbm, seg_out, deg_out,
            src_v, dst_v, rows_v, ones_v, acc_sh, deg_sh, sem):
    c = lax.axis_index("c")
    s = lax.axis_index("s")
    w = s * NC + c

    pltpu.sync_copy(z2_hbm.at[pl.ds(s * ROWS_PT, ROWS_PT)],
                    acc_sh.at[pl.ds(s * ROWS_PT, ROWS_PT)])

    @pl.when(s == 0)
    def _():
        pltpu.sync_copy(z2_hbm.at[pl.ds(NS * ROWS_PT, ROWS_TAIL)],
                        acc_sh.at[pl.ds(NS * ROWS_PT, ROWS_TAIL)])
        pltpu.sync_copy(z1_hbm, deg_sh)

    for i in range(CHUNK // LANES):
        ones_v[pl.ds(i * LANES, LANES)] = jnp.ones((LANES,), jnp.float32)

    plsc.subcore_barrier()

    n_full = NCHUNKS // NW
    n_j = n_full + jnp.where(w < NCHUNKS - n_full * NW, 1, 0)

    def body(j, carry):
        base = (j * NW + w) * CHUNK
        pltpu.sync_copy(src_hbm.at[pl.ds(base, CHUNK)], src_v)
        pltpu.sync_copy(dst_hbm.at[pl.ds(base, CHUNK)], dst_v)
        pltpu.async_copy(y_hbm.at[src_v], rows_v, sem).wait()
        pltpu.sync_copy(rows_v, acc_sh.at[dst_v], add=True)
        pltpu.sync_copy(ones_v, deg_sh.at[dst_v], add=True)
        return carry

    lax.fori_loop(0, n_j, body, 0)

    plsc.subcore_barrier()

    pltpu.sync_copy(acc_sh.at[pl.ds(s * ROWS_PT, ROWS_PT)],
                    seg_out.at[c, pl.ds(s * ROWS_PT, ROWS_PT)])

    @pl.when(s == 0)
    def _():
        pltpu.sync_copy(acc_sh.at[pl.ds(NS * ROWS_PT, ROWS_TAIL)],
                        seg_out.at[c, pl.ds(NS * ROWS_PT, ROWS_TAIL)])
        pltpu.sync_copy(deg_sh, deg_out.at[c])


def _post_body(h_ref, seg_ref, deg_ref, o_ref):
    ssum = seg_ref[0] + seg_ref[1]
    dsum = deg_ref[0] + deg_ref[1]
    o_ref[...] = h_ref[...] + ssum / jnp.maximum(dsum, 1.0)


@jax.jit
def _post(h_self, seg, deg3):
    return pl.pallas_call(
        _post_body,
        grid=(N // _ROWS_BLK,),
        in_specs=[
            pl.BlockSpec((_ROWS_BLK, D), lambda i: (i, 0)),
            pl.BlockSpec((NC, _ROWS_BLK, D), lambda i: (0, i, 0)),
            pl.BlockSpec((NC, _ROWS_BLK, 1), lambda i: (0, i, 0)),
        ],
        out_specs=pl.BlockSpec((_ROWS_BLK, D), lambda i: (i, 0)),
        out_shape=jax.ShapeDtypeStruct((N, D), jnp.float32),
    )(h_self, seg, deg3)


def kernel(x, edge_index, W_neigh, W_self, b_self):
    src = edge_index[0]
    dst = edge_index[1]
    y, h_self = _pre(x, W_neigh, W_self, b_self.reshape(1, D))
    z2 = jnp.zeros((N, D), jnp.float32)
    z1 = jnp.zeros((N,), jnp.float32)
    seg, deg = _sc_seg(y, src, dst, z2, z1)
    return _post(h_self, seg, deg.reshape(NC, N, 1))
